# baseline (speedup 1.0000x reference)
import functools

import jax
import jax.numpy as jnp
from jax import lax
from jax.experimental import pallas as pl
from jax.experimental.pallas import tpu as pltpu
from jax.experimental.pallas import tpu_sc as plsc

B = 4096
L = 200
D = 32
N = B * L
NC = 2
NS = 16
NW = NC * NS
PER_W = N // NW
R = 1600
NCH = PER_W // R
NBUF = 2

K = L * D
KROW = K // 128
TCB = 256
TGRID = B // TCB


def _fire(x_hbm, tab_hbm, idx_v, rows_v, sems, g, base):
    buf = g % NBUF
    off = base + g * R
    pltpu.sync_copy(x_hbm.at[pl.ds(off, R)], idx_v.at[buf])
    pltpu.async_copy(tab_hbm.at[idx_v.at[buf]], rows_v.at[buf], sems.at[buf])


def _emb_body(x_hbm, tab_hbm, out_hbm, idx_v, rows_v, sems):
    wid = lax.axis_index("s") * NC + lax.axis_index("c")
    base = wid * PER_W

    _fire(x_hbm, tab_hbm, idx_v, rows_v, sems, 0, base)

    def chunk_body(g, carry):
        buf = g % NBUF

        @pl.when(g + 1 < NCH)
        def _():
            _fire(x_hbm, tab_hbm, idx_v, rows_v, sems, g + 1, base)

        pltpu.make_async_copy(
            tab_hbm.at[idx_v.at[buf]], rows_v.at[buf], sems.at[buf]).wait()

        off = base + g * R
        pltpu.sync_copy(rows_v.at[buf], out_hbm.at[pl.ds(off, R)])
        return carry

    lax.fori_loop(0, NCH, chunk_body, 0)


V = 1000000
VB = 1984


def _tc_pre_body(tabt_ref, out_ref):
    t = tabt_ref[...]
    r = lax.broadcasted_iota(jnp.int32, (D, D), 0)
    c = lax.broadcasted_iota(jnp.int32, (D, D), 1)
    eye = jnp.where(r == c, 1.0, 0.0).astype(jnp.float32)
    u = lax.dot_general(t, eye, (((0,), (0,)), ((), ())),
                        precision=lax.Precision.HIGHEST,
                        preferred_element_type=jnp.float32)
    u3 = u.reshape(VB, 4, D)
    for k in range(4):
        out_ref[:, D * k:D * (k + 1)] = u3[:, k, :]


def _tc_body(tok_ref, pos_ref, out_ref):
    x = tok_ref[...].reshape(TCB, KROW, 128)
    p = pos_ref[...]
    for r in range(KROW):
        out_ref[pl.ds(r * 128, 128), :] = (
            x[:, r, :].T + p[r * 128:(r + 1) * 128][:, None])


@jax.jit
def _emb(x_flat, tabt, pos_flat):
    tab_pack = pl.pallas_call(
        _tc_pre_body,
        grid=(pl.cdiv(V, 4 * VB),),
        in_specs=[pl.BlockSpec((D, 4 * VB), lambda i: (0, i))],
        out_specs=pl.BlockSpec((VB, 128), lambda i: (i, 0)),
        out_shape=jax.ShapeDtypeStruct((V // 4, 128), jnp.float32),
    )(tabt)
    table = tab_pack.reshape(V, D)

    mesh = plsc.VectorSubcoreMesh(core_axis_name="c", subcore_axis_name="s")
    tok = pl.kernel(
        _emb_body,
        out_type=jax.ShapeDtypeStruct((N, D), jnp.float32),
        mesh=mesh,
        compiler_params=pltpu.CompilerParams(use_tc_tiling_on_sc=False),
        scratch_types=[
            pltpu.VMEM((NBUF, R), jnp.int32),
            pltpu.VMEM((NBUF, R, D), jnp.float32),
            pltpu.SemaphoreType.DMA((NBUF,)),
        ],
    )(x_flat, table)

    tok2 = tok.reshape(N * D // 128, 128)
    out2 = pl.pallas_call(
        _tc_body,
        grid=(TGRID,),
        in_specs=[
            pl.BlockSpec((TCB * KROW, 128), lambda i: (i, 0)),
            pl.BlockSpec((K,), lambda i: (0,)),
        ],
        out_specs=pl.BlockSpec((K, TCB), lambda i: (0, i)),
        out_shape=jax.ShapeDtypeStruct((K, B), jnp.float32),
    )(tok2, pos_flat)
    return out2


def kernel(x, embedding_table, possitional_emb):
    out2 = _emb(x.reshape(-1).astype(jnp.int32), embedding_table.T,
                possitional_emb.reshape(-1))
    return out2.reshape(L, D, B).transpose(2, 0, 1)


# speedup vs baseline: 1.4862x; 1.4862x over previous
import functools

import jax
import jax.numpy as jnp
from jax import lax
from jax.experimental import pallas as pl
from jax.experimental.pallas import tpu as pltpu
from jax.experimental.pallas import tpu_sc as plsc

B = 4096
L = 200
D = 32
N = B * L
NC = 2
NS = 16
NW = NC * NS
PER_W = N // NW
R = 1600
NCH = PER_W // R
NBUF = 2

K = L * D
KROW = K // 128
TCB = 256
TGRID = B // TCB


def _fire(x_hbm, tab_hbm, idx_v, rows_v, sems, g, base):
    buf = g % NBUF
    off = base + g * R
    pltpu.sync_copy(x_hbm.at[pl.ds(off, R)], idx_v.at[buf])
    pltpu.async_copy(tab_hbm.at[idx_v.at[buf]], rows_v.at[buf], sems.at[buf])


def _emb_body(x_hbm, tab_hbm, out_hbm, idx_v, rows_v, sems):
    wid = lax.axis_index("s") * NC + lax.axis_index("c")
    base = wid * PER_W

    _fire(x_hbm, tab_hbm, idx_v, rows_v, sems, 0, base)

    def chunk_body(g, carry):
        buf = g % NBUF

        @pl.when(g + 1 < NCH)
        def _():
            _fire(x_hbm, tab_hbm, idx_v, rows_v, sems, g + 1, base)

        pltpu.make_async_copy(
            tab_hbm.at[idx_v.at[buf]], rows_v.at[buf], sems.at[buf]).wait()

        off = base + g * R
        pltpu.sync_copy(rows_v.at[buf], out_hbm.at[pl.ds(off, R)])
        return carry

    lax.fori_loop(0, NCH, chunk_body, 0)


V = 1000000
VB = 1984


def _tc_pre_body(tabt_ref, out_ref):
    t = tabt_ref[...]
    r = lax.broadcasted_iota(jnp.int32, (D, D), 0)
    c = lax.broadcasted_iota(jnp.int32, (D, D), 1)
    eye = jnp.where(r == c, 1.0, 0.0).astype(jnp.float32)
    u = lax.dot_general(t, eye, (((0,), (0,)), ((), ())),
                        preferred_element_type=jnp.float32)
    u3 = u.reshape(VB, 4, D)
    for k in range(4):
        out_ref[:, D * k:D * (k + 1)] = u3[:, k, :]


def _tc_body(tok_ref, pos_ref, out_ref):
    x = tok_ref[...].reshape(TCB, KROW, 128)
    p = pos_ref[...]
    for r in range(KROW):
        out_ref[pl.ds(r * 128, 128), :] = (
            x[:, r, :].T + p[r * 128:(r + 1) * 128][:, None])


@jax.jit
def _emb(x_flat, tabt, pos_flat):
    tab_pack = pl.pallas_call(
        _tc_pre_body,
        grid=(pl.cdiv(V, 4 * VB),),
        in_specs=[pl.BlockSpec((D, 4 * VB), lambda i: (0, i))],
        out_specs=pl.BlockSpec((VB, 128), lambda i: (i, 0)),
        out_shape=jax.ShapeDtypeStruct((V // 4, 128), jnp.float32),
    )(tabt)
    table = tab_pack.reshape(V, D)

    mesh = plsc.VectorSubcoreMesh(core_axis_name="c", subcore_axis_name="s")
    tok = pl.kernel(
        _emb_body,
        out_type=jax.ShapeDtypeStruct((N, D), jnp.float32),
        mesh=mesh,
        compiler_params=pltpu.CompilerParams(use_tc_tiling_on_sc=False),
        scratch_types=[
            pltpu.VMEM((NBUF, R), jnp.int32),
            pltpu.VMEM((NBUF, R, D), jnp.float32),
            pltpu.SemaphoreType.DMA((NBUF,)),
        ],
    )(x_flat, table)

    tok2 = tok.reshape(N * D // 128, 128)
    out2 = pl.pallas_call(
        _tc_body,
        grid=(TGRID,),
        in_specs=[
            pl.BlockSpec((TCB * KROW, 128), lambda i: (i, 0)),
            pl.BlockSpec((K,), lambda i: (0,)),
        ],
        out_specs=pl.BlockSpec((K, TCB), lambda i: (0, i)),
        out_shape=jax.ShapeDtypeStruct((K, B), jnp.float32),
    )(tok2, pos_flat)
    return out2


def kernel(x, embedding_table, possitional_emb):
    out2 = _emb(x.reshape(-1).astype(jnp.int32), embedding_table.T,
                possitional_emb.reshape(-1))
    return out2.reshape(L, D, B).transpose(2, 0, 1)


# speedup vs baseline: 1.5907x; 1.0703x over previous
import functools

import jax
import jax.numpy as jnp
from jax import lax
from jax.experimental import pallas as pl
from jax.experimental.pallas import tpu as pltpu
from jax.experimental.pallas import tpu_sc as plsc

B = 4096
L = 200
D = 32
N = B * L
NC = 2
NS = 16
NW = NC * NS
PER_W = N // NW
R = 1600
NCH = PER_W // R
NBUF = 2

K = L * D
KROW = K // 128
TCB = 256
TGRID = B // TCB


def _fire(x_hbm, tab_hbm, idx_v, rows_v, sems, g, base):
    buf = g % NBUF
    off = base + g * R
    pltpu.sync_copy(x_hbm.at[pl.ds(off, R)], idx_v.at[buf])
    pltpu.async_copy(tab_hbm.at[idx_v.at[buf]], rows_v.at[buf], sems.at[buf])


def _emb_body(x_hbm, tab_hbm, out_hbm, idx_v, rows_v, sems):
    wid = lax.axis_index("s") * NC + lax.axis_index("c")
    base = wid * PER_W

    _fire(x_hbm, tab_hbm, idx_v, rows_v, sems, 0, base)

    def chunk_body(g, carry):
        buf = g % NBUF

        @pl.when(g + 1 < NCH)
        def _():
            _fire(x_hbm, tab_hbm, idx_v, rows_v, sems, g + 1, base)

        pltpu.make_async_copy(
            tab_hbm.at[idx_v.at[buf]], rows_v.at[buf], sems.at[buf]).wait()

        off = base + g * R
        pltpu.sync_copy(rows_v.at[buf], out_hbm.at[pl.ds(off, R)])
        return carry

    lax.fori_loop(0, NCH, chunk_body, 0)


V = 1000000
VB = 3968


def _tc_pre_body(tabt_ref, out_ref):
    u3 = tabt_ref[...].T.reshape(VB, 4, D)
    for k in range(4):
        out_ref[:, D * k:D * (k + 1)] = u3[:, k, :]


def _tc_body(tok_ref, pos_ref, out_ref):
    x = tok_ref[...].reshape(TCB, KROW, 128)
    p = pos_ref[...]
    for r in range(KROW):
        out_ref[pl.ds(r * 128, 128), :] = (
            x[:, r, :].T + p[r * 128:(r + 1) * 128][:, None])


@jax.jit
def _emb(x_flat, tabt, pos_flat):
    tab_pack = pl.pallas_call(
        _tc_pre_body,
        grid=(pl.cdiv(V, 4 * VB),),
        in_specs=[pl.BlockSpec((D, 4 * VB), lambda i: (0, i))],
        out_specs=pl.BlockSpec((VB, 128), lambda i: (i, 0)),
        out_shape=jax.ShapeDtypeStruct((V // 4, 128), jnp.float32),
    )(tabt)
    table = tab_pack.reshape(V, D)

    mesh = plsc.VectorSubcoreMesh(core_axis_name="c", subcore_axis_name="s")
    tok = pl.kernel(
        _emb_body,
        out_type=jax.ShapeDtypeStruct((N, D), jnp.float32),
        mesh=mesh,
        compiler_params=pltpu.CompilerParams(use_tc_tiling_on_sc=False),
        scratch_types=[
            pltpu.VMEM((NBUF, R), jnp.int32),
            pltpu.VMEM((NBUF, R, D), jnp.float32),
            pltpu.SemaphoreType.DMA((NBUF,)),
        ],
    )(x_flat, table)

    tok2 = tok.reshape(N * D // 128, 128)
    out2 = pl.pallas_call(
        _tc_body,
        grid=(TGRID,),
        in_specs=[
            pl.BlockSpec((TCB * KROW, 128), lambda i: (i, 0)),
            pl.BlockSpec((K,), lambda i: (0,)),
        ],
        out_specs=pl.BlockSpec((K, TCB), lambda i: (0, i)),
        out_shape=jax.ShapeDtypeStruct((K, B), jnp.float32),
    )(tok2, pos_flat)
    return out2


def kernel(x, embedding_table, possitional_emb):
    out2 = _emb(x.reshape(-1).astype(jnp.int32), embedding_table.T,
                possitional_emb.reshape(-1))
    return out2.reshape(L, D, B).transpose(2, 0, 1)
